# baseline (speedup 1.0000x reference)
import math

import jax
import jax.numpy as jnp
from jax.experimental import pallas as pl
from jax.experimental.pallas import tpu as pltpu

B, L, LV = 16, 256, 4
D = 64
HID = 256
PHID = 128
PREP = 16
MIX = 0.1
LOG_LIK_C = math.log(math.sqrt(2.0 * math.pi))
OBS_STD = 0.01
LOG_S2 = math.log(OBS_STD)
INV_2S2 = 1.0 / (2.0 * OBS_STD * OBS_STD)
LVP = 8
HK = (PREP * D) // 2


def _mm(x, w):
    return jax.lax.dot_general(x, w, (((1,), (0,)), ((), ())),
                               preferred_element_type=jnp.float32)


def _fwd_kernel(cov_ref, covW1_ref, covb1_ref, covW2_ref, covb2_ref,
                Whr_ref, Whz_ref, Whh_ref, Wgh_ref, bgh_ref,
                pW1_ref, pb1_ref, pW2_ref, pb2_ref,
                w0f_ref, w1f_ref, w2f_ref, w3f_ref, bpf_ref,
                Wih_a_ref, Wih_b_ref, bih_ref,
                X_ref, M_ref, dtm_ref, dtv_ref,
                h_ref, l1_ref, l2_ref, Hall_ref):
    a = jnp.maximum(cov_ref[...] @ covW1_ref[...] + covb1_ref[...], 0.0)
    h0 = jnp.tanh(a @ covW2_ref[...] + covb2_ref[...])

    Whr = Whr_ref[...]
    Whz = Whz_ref[...]
    Whh = Whh_ref[...]
    Wgh = Wgh_ref[...]
    bgh = bgh_ref[...]
    pW1 = pW1_ref[...]
    pb1 = pb1_ref[...]
    pW2 = pW2_ref[...]
    pb2 = pb2_ref[...]
    w0f = w0f_ref[...]
    w1f = w1f_ref[...]
    w2f = w2f_ref[...]
    w3f = w3f_ref[...]
    bpf = bpf_ref[...]
    Wih_a = Wih_a_ref[...]
    Wih_b = Wih_b_ref[...]
    bih = bih_ref[...]
    two_c = 2.0 * LOG_LIK_C

    def step(t, carry):
        h, pr, pz, acc1 = carry
        Xt = X_ref[t]
        Mt = M_ref[t]
        rep = lambda x: jnp.repeat(x, PREP, axis=1)
        Mrep = rep(Mt)
        base = rep(Xt) * w0f + bpf
        r = jax.nn.sigmoid(pr)
        z = jax.nn.sigmoid(pz)
        u = jnp.tanh(_mm(r * h, Whh))
        h = h + (1.0 - z) * (u - h) * dtm_ref[t]
        a1 = jnp.maximum(_mm(h, pW1) + pb1, 0.0)
        gh = _mm(h, Wgh) + bgh
        p = _mm(a1, pW2) + pb2
        mean = p[:, :D]
        logvar = p[:, D:]
        error = (Xt - mean) * jnp.exp(-0.5 * logvar)
        acc1 = acc1 + (error * error + logvar + two_c) * Mt
        pre = base + rep(mean) * w1f + rep(logvar) * w2f + rep(error) * w3f
        gi = Mrep * jnp.maximum(pre, 0.0)
        gib = gi.astype(jnp.bfloat16)
        gg = _mm(gib[:, :HK], Wih_a) + _mm(gib[:, HK:], Wih_b) + bih
        rg = jax.nn.sigmoid(gg[:, :HID] + gh[:, :HID])
        zg = jax.nn.sigmoid(gg[:, HID:2 * HID] + gh[:, HID:2 * HID])
        n = jnp.tanh(gg[:, 2 * HID:] + rg * gh[:, 2 * HID:])
        h = (1.0 - zg) * n + zg * h
        Hall_ref[t] = h
        return (h, _mm(h, Whr), _mm(h, Whz), acc1)

    def step2(i, carry):
        return step(2 * i + 1, step(2 * i, carry))

    zero = jnp.zeros((B, D), jnp.float32)
    init = (h0, _mm(h0, Whr), _mm(h0, Whz), zero)
    h, pr, pz, acc1 = jax.lax.fori_loop(0, L // 2, step2, init)

    def vstep(j, carry):
        h, pr, pz = carry
        r = jax.nn.sigmoid(pr)
        z = jax.nn.sigmoid(pz)
        u = jnp.tanh(_mm(r * h, Whh))
        h = h + (1.0 - z) * (u - h) * dtv_ref[j]
        return (h, _mm(h, Whr), _mm(h, Whz))

    hf, _, _ = jax.lax.fori_loop(0, LVP, vstep, (h, pr, pz))
    h_ref[...] = hf

    H = Hall_ref[...].reshape(L * B, HID)
    A2 = jnp.maximum(_mm(H, pW1) + pb1, 0.0)
    P2 = _mm(A2, pW2) + pb2
    m2 = P2[:, :D]
    v2 = P2[:, D:]
    Xf = X_ref[...].reshape(L * B, D)
    Mf = M_ref[...].reshape(L * B, D)
    kl = (LOG_S2 - 0.5) - 0.5 * v2 + (jnp.exp(v2) + jnp.square(m2 - Xf)) * INV_2S2
    l1_ref[...] = jnp.reshape(0.5 * jnp.sum(acc1), (1, 1))
    l2_ref[...] = jnp.reshape(jnp.sum(kl * Mf), (1, 1))


def kernel(times, num_obs, X, M, delta_t, cov, val_times, params):
    p = params
    f32 = jnp.float32
    dtm = jnp.concatenate([times[:, :1], times[:, 1:] - times[:, :-1]], axis=1)
    dtm = dtm.T[:, :, None]
    vt = jnp.concatenate([times[:, -1:], val_times], axis=1)
    dtv = (vt[:, 1:] - vt[:, :-1]).T[:, :, None]
    dtv = jnp.concatenate([dtv, jnp.zeros((LVP - LV, B, 1), f32)], axis=0)

    X3 = X.reshape(B, L, D).transpose(1, 0, 2)
    M3 = M.reshape(B, L, D).transpose(1, 0, 2)

    wp = p['w_prep']
    wf = [wp[:, k, :].reshape(1, D * PREP) for k in range(4)]
    bpf = p['bias_prep'].reshape(1, D * PREP)

    out_shapes = [
        jax.ShapeDtypeStruct((B, HID), f32),
        jax.ShapeDtypeStruct((1, 1), f32),
        jax.ShapeDtypeStruct((1, 1), f32),
    ]
    h, l1, l2 = pl.pallas_call(
        _fwd_kernel,
        out_shape=out_shapes,
        scratch_shapes=[pltpu.VMEM((L, B, HID), f32)],
    )(
        cov, p['cov_W1'], p['cov_b1'][None, :], p['cov_W2'], p['cov_b2'][None, :],
        p['ode_Whr'], p['ode_Whz'], p['ode_Whh'],
        p['gru_Whh'], p['gru_bhh'][None, :],
        p['p_W1'], p['p_b1'][None, :], p['p_W2'], p['p_b2'][None, :],
        wf[0], wf[1], wf[2], wf[3], bpf,
        p['gru_Wih'][:HK, :].astype(jnp.bfloat16),
        p['gru_Wih'][HK:, :].astype(jnp.bfloat16), p['gru_bih'][None, :],
        X3, M3, dtm, dtv)
    l1 = l1[0, 0]
    l2 = l2[0, 0]
    loss = l1 + MIX * l2
    nll = l1 / (L * B * D)
    return h, loss, nll, l1, l2


# speedup vs baseline: 1.7617x; 1.7617x over previous
import math

import jax
import jax.numpy as jnp
from jax.experimental import pallas as pl
from jax.experimental.pallas import tpu as pltpu

B, L, LV = 16, 256, 4
D = 64
HID = 256
PHID = 128
PREP = 16
MIX = 0.1
LOG_LIK_C = math.log(math.sqrt(2.0 * math.pi))
OBS_STD = 0.01
LOG_S2 = math.log(OBS_STD)
INV_2S2 = 1.0 / (2.0 * OBS_STD * OBS_STD)
LVP = 8


def _fwd_kernel(cov_ref, covW1_ref, covb1_ref, covW2_ref, covb2_ref,
                Wrz_ref, Whh_ref, Wgh_ref, bgh_ref,
                pW1_ref, pb1_ref, pW2_ref, pb2_ref,
                Axm_ref, Aml_ref, Ae_ref, Wih_ref, bih_ref,
                X_ref, M_ref, dtm_ref, dtv_ref,
                h_ref, l1_ref, l2_ref, Hall_ref):
    a = jnp.maximum(cov_ref[...] @ covW1_ref[...] + covb1_ref[...], 0.0)
    h0 = jnp.tanh(a @ covW2_ref[...] + covb2_ref[...])

    Wrz = Wrz_ref[...]
    Whh = Whh_ref[...]
    Wgh = Wgh_ref[...]
    bgh = bgh_ref[...]
    pW1 = pW1_ref[...]
    pb1 = pb1_ref[...]
    pW2 = pW2_ref[...]
    pb2 = pb2_ref[...]
    Axm = Axm_ref[...]
    Aml = Aml_ref[...]
    Ae = Ae_ref[...]
    Wih = Wih_ref[...]
    bih = bih_ref[...]
    two_c = 2.0 * LOG_LIK_C

    def step(t, carry):
        h, pc, acc1 = carry
        Xt = X_ref[t]
        Mt = M_ref[t]
        base = jnp.concatenate([Xt * Mt, Mt], axis=1) @ Axm
        rz = jax.nn.sigmoid(pc)
        r = rz[:, :HID]
        z = rz[:, HID:]
        u = jnp.tanh((r * h) @ Whh)
        h = h + (1.0 - z) * (u - h) * dtm_ref[t]
        a1 = jnp.maximum(h @ pW1 + pb1, 0.0)
        gh = h @ Wgh + bgh
        p = a1 @ pW2 + pb2
        mean = p[:, :D]
        logvar = p[:, D:]
        error = (Xt - mean) * jnp.exp(-0.5 * logvar)
        acc1 = acc1 + (error * error + logvar + two_c) * Mt
        pre = base + jnp.concatenate([mean * Mt, logvar * Mt], axis=1) @ Aml
        gi = jnp.maximum(pre + (error * Mt) @ Ae, 0.0)
        gg = gi @ Wih + bih
        rg = jax.nn.sigmoid(gg[:, :HID] + gh[:, :HID])
        zg = jax.nn.sigmoid(gg[:, HID:2 * HID] + gh[:, HID:2 * HID])
        n = jnp.tanh(gg[:, 2 * HID:] + rg * gh[:, 2 * HID:])
        h = (1.0 - zg) * n + zg * h
        Hall_ref[t] = h
        pc = h @ Wrz
        return (h, pc, acc1)

    def step4(i, carry):
        carry = step(4 * i + 1, step(4 * i, carry))
        return step(4 * i + 3, step(4 * i + 2, carry))

    zero = jnp.zeros((B, D), jnp.float32)
    pc0 = h0 @ Wrz
    h, pc, acc1 = jax.lax.fori_loop(0, L // 4, step4, (h0, pc0, zero))

    def vstep(j, carry):
        h, pc = carry
        rz = jax.nn.sigmoid(pc)
        r = rz[:, :HID]
        z = rz[:, HID:]
        u = jnp.tanh((r * h) @ Whh)
        h = h + (1.0 - z) * (u - h) * dtv_ref[j]
        return (h, h @ Wrz)

    hf, _ = jax.lax.fori_loop(0, LVP, vstep, (h, pc))
    h_ref[...] = hf

    H = Hall_ref[...].reshape(L * B, HID)
    A2 = jnp.maximum(H @ pW1_ref[...] + pb1_ref[...], 0.0)
    P2 = A2 @ pW2 + pb2
    m2 = P2[:, :D]
    v2 = P2[:, D:]
    Xf = X_ref[...].reshape(L * B, D)
    Mf = M_ref[...].reshape(L * B, D)
    kl = (LOG_S2 - 0.5) - 0.5 * v2 + (jnp.exp(v2) + jnp.square(m2 - Xf)) * INV_2S2
    l1_ref[...] = jnp.reshape(0.5 * jnp.sum(acc1), (1, 1))
    l2_ref[...] = jnp.reshape(jnp.sum(kl * Mf), (1, 1))


def kernel(times, num_obs, X, M, delta_t, cov, val_times, params):
    p = params
    f32 = jnp.float32
    dtm = jnp.concatenate([times[:, :1], times[:, 1:] - times[:, :-1]], axis=1)
    dtm = dtm.T[:, :, None]
    vt = jnp.concatenate([times[:, -1:], val_times], axis=1)
    dtv = (vt[:, 1:] - vt[:, :-1]).T[:, :, None]
    dtv = jnp.concatenate([dtv, jnp.zeros((LVP - LV, B, 1), f32)], axis=0)

    X3 = X.reshape(B, L, D).transpose(1, 0, 2)
    M3 = M.reshape(B, L, D).transpose(1, 0, 2)

    Wrz = jnp.concatenate([p['ode_Whr'], p['ode_Whz']], axis=1)

    eye = jnp.eye(D, dtype=f32)
    wp = p['w_prep']
    blk = [(eye[:, :, None] * wp[None, :, k, :]).reshape(D, D * PREP)
           for k in range(4)]
    bblk = (eye[:, :, None] * p['bias_prep'][None, :, :]).reshape(D, D * PREP)
    Axm = jnp.concatenate([blk[0], bblk], axis=0)
    Aml = jnp.concatenate([blk[1], blk[2]], axis=0)
    Ae = blk[3]

    out_shapes = [
        jax.ShapeDtypeStruct((B, HID), f32),
        jax.ShapeDtypeStruct((1, 1), f32),
        jax.ShapeDtypeStruct((1, 1), f32),
    ]
    h, l1, l2 = pl.pallas_call(
        _fwd_kernel,
        out_shape=out_shapes,
        scratch_shapes=[pltpu.VMEM((L, B, HID), f32)],
    )(
        cov, p['cov_W1'], p['cov_b1'][None, :], p['cov_W2'], p['cov_b2'][None, :],
        Wrz, p['ode_Whh'], p['gru_Whh'], p['gru_bhh'][None, :],
        p['p_W1'], p['p_b1'][None, :], p['p_W2'], p['p_b2'][None, :],
        Axm, Aml, Ae, p['gru_Wih'], p['gru_bih'][None, :],
        X3, M3, dtm, dtv)
    l1 = l1[0, 0]
    l2 = l2[0, 0]
    loss = l1 + MIX * l2
    nll = l1 / (L * B * D)
    return h, loss, nll, l1, l2


# speedup vs baseline: 1.7715x; 1.0056x over previous
import math

import jax
import jax.numpy as jnp
from jax.experimental import pallas as pl
from jax.experimental.pallas import tpu as pltpu

B, L, LV = 16, 256, 4
D = 64
HID = 256
PHID = 128
PREP = 16
MIX = 0.1
LOG_LIK_C = math.log(math.sqrt(2.0 * math.pi))
OBS_STD = 0.01
LOG_S2 = math.log(OBS_STD)
INV_2S2 = 1.0 / (2.0 * OBS_STD * OBS_STD)
LVP = 8


def _fwd_kernel(cov_ref, covW1_ref, covb1_ref, covW2_ref, covb2_ref,
                Wrz_ref, Whh_ref, Wgh_ref, bgh_ref,
                pW1_ref, pb1_ref, pW2_ref, pb2_ref,
                Axm_ref, Aml_ref, Ae_ref, Wih_ref, bih_ref,
                X_ref, M_ref, dtm_ref, dtv_ref,
                h_ref, l1_ref, l2_ref, Hall_ref):
    a = jnp.maximum(cov_ref[...] @ covW1_ref[...] + covb1_ref[...], 0.0)
    h0 = jnp.tanh(a @ covW2_ref[...] + covb2_ref[...])

    Wrz = Wrz_ref[...]
    Whh = Whh_ref[...]
    Wgh = Wgh_ref[...]
    bgh = bgh_ref[...]
    pW1 = pW1_ref[...]
    pb1 = pb1_ref[...]
    pW2 = pW2_ref[...]
    pb2 = pb2_ref[...]
    Axm = Axm_ref[...]
    Aml = Aml_ref[...]
    Ae = Ae_ref[...]
    Wih = Wih_ref[...]
    bih = bih_ref[...]
    two_c = 2.0 * LOG_LIK_C

    def step(t, carry):
        h, pc, acc1 = carry
        Xt = X_ref[t]
        Mt = M_ref[t]
        base = jnp.concatenate([Xt * Mt, Mt], axis=1) @ Axm
        rz = jax.nn.sigmoid(pc)
        r = rz[:, :HID]
        z = rz[:, HID:]
        u = jnp.tanh((r * h) @ Whh)
        h = h + (1.0 - z) * (u - h) * dtm_ref[t]
        a1 = jnp.maximum(h @ pW1 + pb1, 0.0)
        gh = h @ Wgh + bgh
        p = a1 @ pW2 + pb2
        mean = p[:, :D]
        logvar = p[:, D:]
        error = (Xt - mean) * jnp.exp(-0.5 * logvar)
        acc1 = acc1 + (error * error + logvar + two_c) * Mt
        pre = base + jnp.concatenate([mean * Mt, logvar * Mt], axis=1) @ Aml
        gi = jnp.maximum(pre + (error * Mt) @ Ae, 0.0)
        gg = gi @ Wih + bih
        rg = jax.nn.sigmoid(gg[:, :HID] + gh[:, :HID])
        zg = jax.nn.sigmoid(gg[:, HID:2 * HID] + gh[:, HID:2 * HID])
        n = jnp.tanh(gg[:, 2 * HID:] + rg * gh[:, 2 * HID:])
        h = (1.0 - zg) * n + zg * h
        Hall_ref[t] = h
        pc = h @ Wrz
        return (h, pc, acc1)

    def step8(i, carry):
        for k in range(8):
            carry = step(8 * i + k, carry)
        return carry

    zero = jnp.zeros((B, D), jnp.float32)
    pc0 = h0 @ Wrz
    h, pc, acc1 = jax.lax.fori_loop(0, L // 8, step8, (h0, pc0, zero))

    def vstep(j, carry):
        h, pc = carry
        rz = jax.nn.sigmoid(pc)
        r = rz[:, :HID]
        z = rz[:, HID:]
        u = jnp.tanh((r * h) @ Whh)
        h = h + (1.0 - z) * (u - h) * dtv_ref[j]
        return (h, h @ Wrz)

    hf, _ = jax.lax.fori_loop(0, LVP, vstep, (h, pc))
    h_ref[...] = hf

    H = Hall_ref[...].reshape(L * B, HID)
    A2 = jnp.maximum(H @ pW1_ref[...] + pb1_ref[...], 0.0)
    P2 = A2 @ pW2 + pb2
    m2 = P2[:, :D]
    v2 = P2[:, D:]
    Xf = X_ref[...].reshape(L * B, D)
    Mf = M_ref[...].reshape(L * B, D)
    kl = (LOG_S2 - 0.5) - 0.5 * v2 + (jnp.exp(v2) + jnp.square(m2 - Xf)) * INV_2S2
    l1_ref[...] = jnp.reshape(0.5 * jnp.sum(acc1), (1, 1))
    l2_ref[...] = jnp.reshape(jnp.sum(kl * Mf), (1, 1))


def kernel(times, num_obs, X, M, delta_t, cov, val_times, params):
    p = params
    f32 = jnp.float32
    dtm = jnp.concatenate([times[:, :1], times[:, 1:] - times[:, :-1]], axis=1)
    dtm = dtm.T[:, :, None]
    vt = jnp.concatenate([times[:, -1:], val_times], axis=1)
    dtv = (vt[:, 1:] - vt[:, :-1]).T[:, :, None]
    dtv = jnp.concatenate([dtv, jnp.zeros((LVP - LV, B, 1), f32)], axis=0)

    X3 = X.reshape(B, L, D).transpose(1, 0, 2)
    M3 = M.reshape(B, L, D).transpose(1, 0, 2)

    Wrz = jnp.concatenate([p['ode_Whr'], p['ode_Whz']], axis=1)

    eye = jnp.eye(D, dtype=f32)
    wp = p['w_prep']
    blk = [(eye[:, :, None] * wp[None, :, k, :]).reshape(D, D * PREP)
           for k in range(4)]
    bblk = (eye[:, :, None] * p['bias_prep'][None, :, :]).reshape(D, D * PREP)
    Axm = jnp.concatenate([blk[0], bblk], axis=0)
    Aml = jnp.concatenate([blk[1], blk[2]], axis=0)
    Ae = blk[3]

    out_shapes = [
        jax.ShapeDtypeStruct((B, HID), f32),
        jax.ShapeDtypeStruct((1, 1), f32),
        jax.ShapeDtypeStruct((1, 1), f32),
    ]
    h, l1, l2 = pl.pallas_call(
        _fwd_kernel,
        out_shape=out_shapes,
        scratch_shapes=[pltpu.VMEM((L, B, HID), f32)],
    )(
        cov, p['cov_W1'], p['cov_b1'][None, :], p['cov_W2'], p['cov_b2'][None, :],
        Wrz, p['ode_Whh'], p['gru_Whh'], p['gru_bhh'][None, :],
        p['p_W1'], p['p_b1'][None, :], p['p_W2'], p['p_b2'][None, :],
        Axm, Aml, Ae, p['gru_Wih'], p['gru_bih'][None, :],
        X3, M3, dtm, dtv)
    l1 = l1[0, 0]
    l2 = l2[0, 0]
    loss = l1 + MIX * l2
    nll = l1 / (L * B * D)
    return h, loss, nll, l1, l2


# speedup vs baseline: 1.7775x; 1.0034x over previous
import math

import jax
import jax.numpy as jnp
from jax.experimental import pallas as pl
from jax.experimental.pallas import tpu as pltpu

B, L, LV = 16, 256, 4
D = 64
HID = 256
PHID = 128
PREP = 16
MIX = 0.1
LOG_LIK_C = math.log(math.sqrt(2.0 * math.pi))
OBS_STD = 0.01
LOG_S2 = math.log(OBS_STD)
INV_2S2 = 1.0 / (2.0 * OBS_STD * OBS_STD)
LVP = 8


def _fwd_kernel(cov_ref, covW1_ref, covb1_ref, covW2_ref, covb2_ref,
                Wrz_ref, Whh_ref, Wgh_ref, bgh_ref,
                pW1_ref, pb1_ref, pW2_ref, pb2_ref,
                Axm_ref, Aml_ref, Ae_ref, Wih_ref, bih_ref,
                X_ref, M_ref, dtm_ref, dtv_ref,
                h_ref, l1_ref, l2_ref, Hall_ref):
    a = jnp.maximum(cov_ref[...] @ covW1_ref[...] + covb1_ref[...], 0.0)
    h0 = jnp.tanh(a @ covW2_ref[...] + covb2_ref[...])

    Wrz = Wrz_ref[...]
    Whh = Whh_ref[...]
    Wgh = Wgh_ref[...]
    bgh = bgh_ref[...]
    pW1 = pW1_ref[...]
    pb1 = pb1_ref[...]
    pW2 = pW2_ref[...]
    pb2 = pb2_ref[...]
    Axm = Axm_ref[...]
    Aml = Aml_ref[...]
    Ae = Ae_ref[...]
    Wih = Wih_ref[...]
    bih = bih_ref[...]
    two_c = 2.0 * LOG_LIK_C

    def step(t, carry):
        h, pc, acc1 = carry
        Xt = X_ref[t]
        Mt = M_ref[t]
        base = jnp.concatenate([Xt * Mt, Mt], axis=1) @ Axm
        rz = jax.nn.sigmoid(pc)
        r = rz[:, :HID]
        z = rz[:, HID:]
        u = jnp.tanh((r * h) @ Whh)
        h = h + (1.0 - z) * (u - h) * dtm_ref[t]
        a1 = jnp.maximum(h @ pW1 + pb1, 0.0)
        gh = h @ Wgh + bgh
        p = a1 @ pW2 + pb2
        mean = p[:, :D]
        logvar = p[:, D:]
        error = (Xt - mean) * jnp.exp(-0.5 * logvar)
        acc1 = acc1 + (error * error + logvar + two_c) * Mt
        pre = base + jnp.concatenate([mean * Mt, logvar * Mt], axis=1) @ Aml
        gi = jnp.maximum(pre + (error * Mt) @ Ae, 0.0)
        gg = gi @ Wih + bih
        rg = jax.nn.sigmoid(gg[:, :HID] + gh[:, :HID])
        zg = jax.nn.sigmoid(gg[:, HID:2 * HID] + gh[:, HID:2 * HID])
        n = jnp.tanh(gg[:, 2 * HID:] + rg * gh[:, 2 * HID:])
        h = (1.0 - zg) * n + zg * h
        Hall_ref[t] = h
        pc = h @ Wrz
        return (h, pc, acc1)

    def step16(i, carry):
        for k in range(16):
            carry = step(16 * i + k, carry)
        return carry

    zero = jnp.zeros((B, D), jnp.float32)
    pc0 = h0 @ Wrz
    h, pc, acc1 = jax.lax.fori_loop(0, L // 16, step16, (h0, pc0, zero))

    def vstep(j, carry):
        h, pc = carry
        rz = jax.nn.sigmoid(pc)
        r = rz[:, :HID]
        z = rz[:, HID:]
        u = jnp.tanh((r * h) @ Whh)
        h = h + (1.0 - z) * (u - h) * dtv_ref[j]
        return (h, h @ Wrz)

    hf, _ = jax.lax.fori_loop(0, LVP, vstep, (h, pc))
    h_ref[...] = hf

    H = Hall_ref[...].reshape(L * B, HID)
    A2 = jnp.maximum(H @ pW1_ref[...] + pb1_ref[...], 0.0)
    P2 = A2 @ pW2 + pb2
    m2 = P2[:, :D]
    v2 = P2[:, D:]
    Xf = X_ref[...].reshape(L * B, D)
    Mf = M_ref[...].reshape(L * B, D)
    kl = (LOG_S2 - 0.5) - 0.5 * v2 + (jnp.exp(v2) + jnp.square(m2 - Xf)) * INV_2S2
    l1_ref[...] = jnp.reshape(0.5 * jnp.sum(acc1), (1, 1))
    l2_ref[...] = jnp.reshape(jnp.sum(kl * Mf), (1, 1))


def kernel(times, num_obs, X, M, delta_t, cov, val_times, params):
    p = params
    f32 = jnp.float32
    dtm = jnp.concatenate([times[:, :1], times[:, 1:] - times[:, :-1]], axis=1)
    dtm = dtm.T[:, :, None]
    vt = jnp.concatenate([times[:, -1:], val_times], axis=1)
    dtv = (vt[:, 1:] - vt[:, :-1]).T[:, :, None]
    dtv = jnp.concatenate([dtv, jnp.zeros((LVP - LV, B, 1), f32)], axis=0)

    X3 = X.reshape(B, L, D).transpose(1, 0, 2)
    M3 = M.reshape(B, L, D).transpose(1, 0, 2)

    Wrz = jnp.concatenate([p['ode_Whr'], p['ode_Whz']], axis=1)

    eye = jnp.eye(D, dtype=f32)
    wp = p['w_prep']
    blk = [(eye[:, :, None] * wp[None, :, k, :]).reshape(D, D * PREP)
           for k in range(4)]
    bblk = (eye[:, :, None] * p['bias_prep'][None, :, :]).reshape(D, D * PREP)
    Axm = jnp.concatenate([blk[0], bblk], axis=0)
    Aml = jnp.concatenate([blk[1], blk[2]], axis=0)
    Ae = blk[3]

    out_shapes = [
        jax.ShapeDtypeStruct((B, HID), f32),
        jax.ShapeDtypeStruct((1, 1), f32),
        jax.ShapeDtypeStruct((1, 1), f32),
    ]
    h, l1, l2 = pl.pallas_call(
        _fwd_kernel,
        out_shape=out_shapes,
        scratch_shapes=[pltpu.VMEM((L, B, HID), f32)],
    )(
        cov, p['cov_W1'], p['cov_b1'][None, :], p['cov_W2'], p['cov_b2'][None, :],
        Wrz, p['ode_Whh'], p['gru_Whh'], p['gru_bhh'][None, :],
        p['p_W1'], p['p_b1'][None, :], p['p_W2'], p['p_b2'][None, :],
        Axm, Aml, Ae, p['gru_Wih'], p['gru_bih'][None, :],
        X3, M3, dtm, dtv)
    l1 = l1[0, 0]
    l2 = l2[0, 0]
    loss = l1 + MIX * l2
    nll = l1 / (L * B * D)
    return h, loss, nll, l1, l2


# speedup vs baseline: 1.7829x; 1.0030x over previous
import math

import jax
import jax.numpy as jnp
from jax.experimental import pallas as pl
from jax.experimental.pallas import tpu as pltpu

B, L, LV = 16, 256, 4
D = 64
HID = 256
PHID = 128
PREP = 16
MIX = 0.1
LOG_LIK_C = math.log(math.sqrt(2.0 * math.pi))
OBS_STD = 0.01
LOG_S2 = math.log(OBS_STD)
INV_2S2 = 1.0 / (2.0 * OBS_STD * OBS_STD)
LVP = 8


def _fwd_kernel(cov_ref, covW1_ref, covb1_ref, covW2_ref, covb2_ref,
                Wrz_ref, Whh_ref, Wgh_ref, bgh_ref,
                pW1_ref, pb1_ref, pW2_ref, pb2_ref,
                Axm_ref, Aml_ref, Ae_ref, Wih_ref, bih_ref,
                X_ref, M_ref, dtm_ref, dtv_ref,
                h_ref, l1_ref, l2_ref, Hall_ref):
    a = jnp.maximum(cov_ref[...] @ covW1_ref[...] + covb1_ref[...], 0.0)
    h0 = jnp.tanh(a @ covW2_ref[...] + covb2_ref[...])

    Wrz = Wrz_ref[...]
    Whh = Whh_ref[...]
    Wgh = Wgh_ref[...]
    bgh = bgh_ref[...]
    pW1 = pW1_ref[...]
    pb1 = pb1_ref[...]
    pW2 = pW2_ref[...]
    pb2 = pb2_ref[...]
    Axm = Axm_ref[...]
    Aml = Aml_ref[...]
    Ae = Ae_ref[...]
    Wih = Wih_ref[...]
    bih = bih_ref[...]
    two_c = 2.0 * LOG_LIK_C

    def sig(x):
        return 0.5 * jnp.tanh(0.5 * x) + 0.5

    def step(t, carry):
        h, pc, acc1 = carry
        Xt = X_ref[t]
        Mt = M_ref[t]
        base = jnp.concatenate([Xt * Mt, Mt], axis=1) @ Axm
        rz = sig(pc)
        r = rz[:, :HID]
        z = rz[:, HID:]
        u = jnp.tanh((r * h) @ Whh)
        h = h + (1.0 - z) * (u - h) * dtm_ref[t]
        a1 = jnp.maximum(h @ pW1 + pb1, 0.0)
        gh = h @ Wgh + bgh
        p = a1 @ pW2 + pb2
        mean = p[:, :D]
        logvar = p[:, D:]
        error = (Xt - mean) * jnp.exp(-0.5 * logvar)
        acc1 = acc1 + (error * error + logvar + two_c) * Mt
        pre = base + jnp.concatenate([mean * Mt, logvar * Mt], axis=1) @ Aml
        gi = jnp.maximum(pre + (error * Mt) @ Ae, 0.0)
        gg = gi @ Wih + bih
        rg = sig(gg[:, :HID] + gh[:, :HID])
        zg = sig(gg[:, HID:2 * HID] + gh[:, HID:2 * HID])
        n = jnp.tanh(gg[:, 2 * HID:] + rg * gh[:, 2 * HID:])
        h = (1.0 - zg) * n + zg * h
        Hall_ref[t] = h
        pc = h @ Wrz
        return (h, pc, acc1)

    def step16(i, carry):
        for k in range(16):
            carry = step(16 * i + k, carry)
        return carry

    zero = jnp.zeros((B, D), jnp.float32)
    pc0 = h0 @ Wrz
    h, pc, acc1 = jax.lax.fori_loop(0, L // 16, step16, (h0, pc0, zero))

    def vstep(j, carry):
        h, pc = carry
        rz = sig(pc)
        r = rz[:, :HID]
        z = rz[:, HID:]
        u = jnp.tanh((r * h) @ Whh)
        h = h + (1.0 - z) * (u - h) * dtv_ref[j]
        return (h, h @ Wrz)

    hf, _ = jax.lax.fori_loop(0, LVP, vstep, (h, pc))
    h_ref[...] = hf

    H = Hall_ref[...].reshape(L * B, HID)
    A2 = jnp.maximum(H @ pW1_ref[...] + pb1_ref[...], 0.0)
    P2 = A2 @ pW2 + pb2
    m2 = P2[:, :D]
    v2 = P2[:, D:]
    Xf = X_ref[...].reshape(L * B, D)
    Mf = M_ref[...].reshape(L * B, D)
    kl = (LOG_S2 - 0.5) - 0.5 * v2 + (jnp.exp(v2) + jnp.square(m2 - Xf)) * INV_2S2
    l1_ref[...] = jnp.reshape(0.5 * jnp.sum(acc1), (1, 1))
    l2_ref[...] = jnp.reshape(jnp.sum(kl * Mf), (1, 1))


def kernel(times, num_obs, X, M, delta_t, cov, val_times, params):
    p = params
    f32 = jnp.float32
    dtm = jnp.concatenate([times[:, :1], times[:, 1:] - times[:, :-1]], axis=1)
    dtm = dtm.T[:, :, None]
    vt = jnp.concatenate([times[:, -1:], val_times], axis=1)
    dtv = (vt[:, 1:] - vt[:, :-1]).T[:, :, None]
    dtv = jnp.concatenate([dtv, jnp.zeros((LVP - LV, B, 1), f32)], axis=0)

    X3 = X.reshape(B, L, D).transpose(1, 0, 2)
    M3 = M.reshape(B, L, D).transpose(1, 0, 2)

    Wrz = jnp.concatenate([p['ode_Whr'], p['ode_Whz']], axis=1)

    eye = jnp.eye(D, dtype=f32)
    wp = p['w_prep']
    blk = [(eye[:, :, None] * wp[None, :, k, :]).reshape(D, D * PREP)
           for k in range(4)]
    bblk = (eye[:, :, None] * p['bias_prep'][None, :, :]).reshape(D, D * PREP)
    Axm = jnp.concatenate([blk[0], bblk], axis=0)
    Aml = jnp.concatenate([blk[1], blk[2]], axis=0)
    Ae = blk[3]

    out_shapes = [
        jax.ShapeDtypeStruct((B, HID), f32),
        jax.ShapeDtypeStruct((1, 1), f32),
        jax.ShapeDtypeStruct((1, 1), f32),
    ]
    h, l1, l2 = pl.pallas_call(
        _fwd_kernel,
        out_shape=out_shapes,
        scratch_shapes=[pltpu.VMEM((L, B, HID), f32)],
    )(
        cov, p['cov_W1'], p['cov_b1'][None, :], p['cov_W2'], p['cov_b2'][None, :],
        Wrz, p['ode_Whh'], p['gru_Whh'], p['gru_bhh'][None, :],
        p['p_W1'], p['p_b1'][None, :], p['p_W2'], p['p_b2'][None, :],
        Axm, Aml, Ae, p['gru_Wih'], p['gru_bih'][None, :],
        X3, M3, dtm, dtv)
    l1 = l1[0, 0]
    l2 = l2[0, 0]
    loss = l1 + MIX * l2
    nll = l1 / (L * B * D)
    return h, loss, nll, l1, l2


# speedup vs baseline: 1.8020x; 1.0107x over previous
import math

import jax
import jax.numpy as jnp
from jax.experimental import pallas as pl
from jax.experimental.pallas import tpu as pltpu

B, L, LV = 16, 256, 4
D = 64
HID = 256
PHID = 128
PREP = 16
MIX = 0.1
LOG_LIK_C = math.log(math.sqrt(2.0 * math.pi))
OBS_STD = 0.01
LOG_S2 = math.log(OBS_STD)
INV_2S2 = 1.0 / (2.0 * OBS_STD * OBS_STD)
LVP = 4


def _fwd_kernel(cov_ref, covW1_ref, covb1_ref, covW2_ref, covb2_ref,
                Wrz_ref, Whh_ref, Wgh_ref, bgh_ref,
                pW1_ref, pb1_ref, pW2_ref, pb2_ref,
                Axm_ref, Aml_ref, Ae_ref, Wih_ref, bih_ref,
                X_ref, M_ref, dtm_ref, dtv_ref,
                h_ref, l1_ref, l2_ref, Hall_ref):
    a = jnp.maximum(cov_ref[...] @ covW1_ref[...] + covb1_ref[...], 0.0)
    h0 = jnp.tanh(a @ covW2_ref[...] + covb2_ref[...])

    Wrz = Wrz_ref[...]
    Whh = Whh_ref[...]
    Wgh = Wgh_ref[...]
    bgh = bgh_ref[...]
    pW1 = pW1_ref[...]
    pb1 = pb1_ref[...]
    pW2 = pW2_ref[...]
    pb2 = pb2_ref[...]
    Axm = Axm_ref[...]
    Aml = Aml_ref[...]
    Ae = Ae_ref[...]
    Wih = Wih_ref[...]
    bih = bih_ref[...]
    two_c = 2.0 * LOG_LIK_C

    def sig(x):
        return 0.5 * jnp.tanh(0.5 * x) + 0.5

    def step(t, carry):
        h, pc, acc1 = carry
        Xt = X_ref[t]
        Mt = M_ref[t]
        base = jnp.concatenate([Xt * Mt, Mt], axis=1) @ Axm
        rz = sig(pc)
        r = rz[:, :HID]
        z = rz[:, HID:]
        u = jnp.tanh((r * h) @ Whh)
        h = h + (1.0 - z) * (u - h) * dtm_ref[t]
        a1 = jnp.maximum(h @ pW1 + pb1, 0.0)
        gh = h @ Wgh + bgh
        p = a1 @ pW2 + pb2
        mean = p[:, :D]
        logvar = p[:, D:]
        error = (Xt - mean) * jnp.exp(-0.5 * logvar)
        acc1 = acc1 + (error * error + logvar + two_c) * Mt
        pre = base + jnp.concatenate([mean * Mt, logvar * Mt], axis=1) @ Aml
        gi = jnp.maximum(pre + (error * Mt) @ Ae, 0.0)
        gg = gi @ Wih + bih
        rg = sig(gg[:, :HID] + gh[:, :HID])
        zg = sig(gg[:, HID:2 * HID] + gh[:, HID:2 * HID])
        n = jnp.tanh(gg[:, 2 * HID:] + rg * gh[:, 2 * HID:])
        h = (1.0 - zg) * n + zg * h
        Hall_ref[t] = h
        pc = h @ Wrz
        return (h, pc, acc1)

    def step16(i, carry):
        for k in range(16):
            carry = step(16 * i + k, carry)
        return carry

    zero = jnp.zeros((B, D), jnp.float32)
    pc0 = h0 @ Wrz
    h, pc, acc1 = jax.lax.fori_loop(0, L // 16, step16, (h0, pc0, zero))

    def vstep(j, carry):
        h, pc = carry
        rz = sig(pc)
        r = rz[:, :HID]
        z = rz[:, HID:]
        u = jnp.tanh((r * h) @ Whh)
        h = h + (1.0 - z) * (u - h) * dtv_ref[j]
        return (h, h @ Wrz)

    hf, _ = jax.lax.fori_loop(0, LVP, vstep, (h, pc))
    h_ref[...] = hf

    H = Hall_ref[...].reshape(L * B, HID)
    A2 = jnp.maximum(H @ pW1_ref[...] + pb1_ref[...], 0.0)
    P2 = A2 @ pW2 + pb2
    m2 = P2[:, :D]
    v2 = P2[:, D:]
    Xf = X_ref[...].reshape(L * B, D)
    Mf = M_ref[...].reshape(L * B, D)
    kl = (LOG_S2 - 0.5) - 0.5 * v2 + (jnp.exp(v2) + jnp.square(m2 - Xf)) * INV_2S2
    l1_ref[...] = jnp.reshape(0.5 * jnp.sum(acc1), (1, 1))
    l2_ref[...] = jnp.reshape(jnp.sum(kl * Mf), (1, 1))


def kernel(times, num_obs, X, M, delta_t, cov, val_times, params):
    p = params
    f32 = jnp.float32
    dtm = jnp.concatenate([times[:, :1], times[:, 1:] - times[:, :-1]], axis=1)
    dtm = dtm.T[:, :, None]
    vt = jnp.concatenate([times[:, -1:], val_times], axis=1)
    dtv = (vt[:, 1:] - vt[:, :-1]).T[:, :, None]

    X3 = X.reshape(B, L, D).transpose(1, 0, 2)
    M3 = M.reshape(B, L, D).transpose(1, 0, 2)

    Wrz = jnp.concatenate([p['ode_Whr'], p['ode_Whz']], axis=1)

    eye = jnp.eye(D, dtype=f32)
    wp = p['w_prep']
    blk = [(eye[:, :, None] * wp[None, :, k, :]).reshape(D, D * PREP)
           for k in range(4)]
    bblk = (eye[:, :, None] * p['bias_prep'][None, :, :]).reshape(D, D * PREP)
    Axm = jnp.concatenate([blk[0], bblk], axis=0)
    Aml = jnp.concatenate([blk[1], blk[2]], axis=0)
    Ae = blk[3]

    out_shapes = [
        jax.ShapeDtypeStruct((B, HID), f32),
        jax.ShapeDtypeStruct((1, 1), f32),
        jax.ShapeDtypeStruct((1, 1), f32),
    ]
    h, l1, l2 = pl.pallas_call(
        _fwd_kernel,
        out_shape=out_shapes,
        scratch_shapes=[pltpu.VMEM((L, B, HID), f32)],
    )(
        cov, p['cov_W1'], p['cov_b1'][None, :], p['cov_W2'], p['cov_b2'][None, :],
        Wrz, p['ode_Whh'], p['gru_Whh'], p['gru_bhh'][None, :],
        p['p_W1'], p['p_b1'][None, :], p['p_W2'], p['p_b2'][None, :],
        Axm, Aml, Ae, p['gru_Wih'], p['gru_bih'][None, :],
        X3, M3, dtm, dtv)
    l1 = l1[0, 0]
    l2 = l2[0, 0]
    loss = l1 + MIX * l2
    nll = l1 / (L * B * D)
    return h, loss, nll, l1, l2
